# baseline (speedup 1.0000x reference)
import functools

import jax
import jax.numpy as jnp
from jax.experimental import pallas as pl
from jax.experimental.pallas import tpu as pltpu


def _pd_kernel(q_ref, ctx_ref, msk_ref, wattn_ref, wsw_ref, bsw_ref,
               out_ref, wgt_ref, sw_ref, scr_ref, *, D, bb, sc, ns):
  s_idx = pl.program_id(1)

  for j in range(bb):
    q32 = q_ref[j]
    qb = q32.astype(jnp.bfloat16)

    ks = ctx_ref[j, pl.ds(s_idx * sc, sc), :].astype(jnp.bfloat16)
    sch = jax.lax.dot_general(qb, ks, (((1,), (1,)), ((), ())),
                              preferred_element_type=jnp.float32)
    sch = jnp.where(msk_ref[j] != 0.0, sch, jnp.float32(-1e30))
    lane0 = pl.multiple_of(s_idx * sc, 128)
    scr_ref[j, :, pl.ds(lane0, sc)] = sch

    @pl.when(s_idx == ns - 1)
    def _finalize(j=j, q32=q32, qb=qb):
      scores = scr_ref[j]
      m = jnp.max(scores, axis=-1, keepdims=True)
      p = jnp.exp(scores - m)
      l = jnp.sum(p, axis=-1, keepdims=True)
      inv_l = pl.reciprocal(l, approx=False)
      wgt_ref[j] = p * inv_l

      cb = ctx_ref[j].astype(jnp.bfloat16)
      attn = jax.lax.dot_general(p.astype(jnp.bfloat16), cb,
                                 (((1,), (0,)), ((), ())),
                                 preferred_element_type=jnp.float32) * inv_l

      w_top = wattn_ref[:, :D].astype(jnp.bfloat16)
      w_bot = wattn_ref[:, D:].astype(jnp.bfloat16)
      h = (jax.lax.dot_general(qb, w_top, (((1,), (1,)), ((), ())),
                               preferred_element_type=jnp.float32) +
           jax.lax.dot_general(attn.astype(jnp.bfloat16), w_bot,
                               (((1,), (1,)), ((), ())),
                               preferred_element_type=jnp.float32))
      out = jnp.tanh(h)
      out_ref[j] = out.astype(out_ref.dtype)

      z = (jax.lax.dot_general(wsw_ref[0:1, :D], q32, (((1,), (1,)), ((), ())),
                               preferred_element_type=jnp.float32) +
           jax.lax.dot_general(wsw_ref[0:1, D:], out, (((1,), (1,)), ((), ())),
                               preferred_element_type=jnp.float32) +
           bsw_ref[...])
      sw_ref[j] = jax.nn.sigmoid(z)


def _round_up(x, m):
  return ((x + m - 1) // m) * m


def kernel(inp, context, atten_mask, w_attn, w_switch, b_switch):
  B, T, D = inp.shape
  _, S, _ = context.shape

  b_sw = jnp.asarray(b_switch, jnp.float32).reshape(1, 1)

  bb = 2 if B % 2 == 0 else 1
  tm = min(512, _round_up(T, 8))
  t_pad, s_pad = _round_up(T, tm), _round_up(S, 128)
  nt = t_pad // tm
  sc = s_pad // 2 if s_pad % 1024 == 0 else s_pad
  ns = s_pad // sc

  q = jnp.asarray(inp, jnp.float32)
  ctx = jnp.asarray(context, jnp.float32)
  msk = jnp.asarray(atten_mask, jnp.float32)
  if t_pad != T:
    q = jnp.pad(q, ((0, 0), (0, t_pad - T), (0, 0)))
    msk = jnp.pad(msk, ((0, 0), (0, t_pad - T), (0, 0)))
  if s_pad != S:
    ctx = jnp.pad(ctx, ((0, 0), (0, s_pad - S), (0, 0)))
    msk = jnp.pad(msk, ((0, 0), (0, 0), (0, s_pad - S)))

  out_shapes = (
      jax.ShapeDtypeStruct((B, t_pad, D), jnp.float32),
      jax.ShapeDtypeStruct((B, t_pad, s_pad), jnp.float32),
      jax.ShapeDtypeStruct((B, 1, t_pad), jnp.float32),
  )

  kfn = functools.partial(_pd_kernel, D=D, bb=bb, sc=sc, ns=ns)

  out, weight, switch = pl.pallas_call(
      kfn,
      out_shape=out_shapes,
      grid=(B // bb, ns),
      in_specs=[
          pl.BlockSpec((bb, tm, D), lambda b, s: (b, 0, 0)),
          pl.BlockSpec((bb, s_pad, D), lambda b, s: (b, 0, 0)),
          pl.BlockSpec((bb, tm, sc), lambda b, s: (b, 0, s)),
          pl.BlockSpec((D, 2 * D), lambda b, s: (0, 0)),
          pl.BlockSpec((1, 2 * D), lambda b, s: (0, 0)),
          pl.BlockSpec((1, 1), lambda b, s: (0, 0)),
      ],
      out_specs=[
          pl.BlockSpec((bb, tm, D), lambda b, s: (b, 0, 0)),
          pl.BlockSpec((bb, tm, s_pad), lambda b, s: (b, 0, 0)),
          pl.BlockSpec((bb, 1, tm), lambda b, s: (b, 0, 0)),
      ],
      scratch_shapes=[pltpu.VMEM((bb, tm, s_pad), jnp.float32)],
      compiler_params=pltpu.CompilerParams(
          dimension_semantics=("parallel", "arbitrary"),
          vmem_limit_bytes=56 << 20),
  )(q, ctx, msk, jnp.asarray(w_attn, jnp.float32),
    jnp.asarray(w_switch, jnp.float32), b_sw)

  switch = switch.reshape(B, t_pad, 1)
  return out[:, :T, :], weight[:, :T, :S], switch[:, :T, :]


# speedup vs baseline: 1.4407x; 1.4407x over previous
import functools

import jax
import jax.numpy as jnp
from jax.experimental import pallas as pl
from jax.experimental.pallas import tpu as pltpu


def _pd_kernel(q_ref, ctx_ref, msk_ref, wattn_ref, wsw_ref, bsw_ref,
               out_ref, wgt_ref, sw_ref, *, D, bb):
  w_top = wattn_ref[:, :D].astype(jnp.bfloat16)
  w_bot = wattn_ref[:, D:].astype(jnp.bfloat16)

  for j in range(bb):
    q32 = q_ref[j]
    qb = q32.astype(jnp.bfloat16)
    cb = ctx_ref[j].astype(jnp.bfloat16)

    scores = jax.lax.dot_general(qb, cb, (((1,), (1,)), ((), ())),
                                 preferred_element_type=jnp.float32)
    scores = jnp.where(msk_ref[j] != 0.0, scores, jnp.float32(-1e30))

    m = jnp.max(scores, axis=-1, keepdims=True)
    p = jnp.exp(scores - m)
    l = jnp.sum(p, axis=-1, keepdims=True)
    inv_l = pl.reciprocal(l, approx=False)
    wgt_ref[j] = p * inv_l

    attn = jax.lax.dot_general(p.astype(jnp.bfloat16), cb,
                               (((1,), (0,)), ((), ())),
                               preferred_element_type=jnp.float32) * inv_l

    h = (jax.lax.dot_general(qb, w_top, (((1,), (1,)), ((), ())),
                             preferred_element_type=jnp.float32) +
         jax.lax.dot_general(attn.astype(jnp.bfloat16), w_bot,
                             (((1,), (1,)), ((), ())),
                             preferred_element_type=jnp.float32))
    out = jnp.tanh(h)
    out_ref[j] = out.astype(out_ref.dtype)

    z = (jax.lax.dot_general(wsw_ref[0:1, :D], q32, (((1,), (1,)), ((), ())),
                             preferred_element_type=jnp.float32) +
         jax.lax.dot_general(wsw_ref[0:1, D:], out, (((1,), (1,)), ((), ())),
                             preferred_element_type=jnp.float32) +
         bsw_ref[...])
    sw_ref[j] = jax.nn.sigmoid(z)


def _round_up(x, m):
  return ((x + m - 1) // m) * m


def kernel(inp, context, atten_mask, w_attn, w_switch, b_switch):
  B, T, D = inp.shape
  _, S, _ = context.shape

  b_sw = jnp.asarray(b_switch, jnp.float32).reshape(1, 1)

  bb = 2 if B % 2 == 0 else 1
  tm = min(512, _round_up(T, 8))
  t_pad, s_pad = _round_up(T, tm), _round_up(S, 128)
  nt = t_pad // tm

  q = jnp.asarray(inp, jnp.float32)
  ctx = jnp.asarray(context, jnp.float32)
  msk = jnp.asarray(atten_mask, jnp.float32)
  if t_pad != T:
    q = jnp.pad(q, ((0, 0), (0, t_pad - T), (0, 0)))
    msk = jnp.pad(msk, ((0, 0), (0, t_pad - T), (0, 0)))
  if s_pad != S:
    ctx = jnp.pad(ctx, ((0, 0), (0, s_pad - S), (0, 0)))
    msk = jnp.pad(msk, ((0, 0), (0, 0), (0, s_pad - S)))

  out_shapes = (
      jax.ShapeDtypeStruct((B, t_pad, D), jnp.float32),
      jax.ShapeDtypeStruct((B, t_pad, s_pad), jnp.float32),
      jax.ShapeDtypeStruct((B, 1, t_pad), jnp.float32),
  )

  kfn = functools.partial(_pd_kernel, D=D, bb=bb)

  out, weight, switch = pl.pallas_call(
      kfn,
      out_shape=out_shapes,
      grid=(B // bb, nt),
      in_specs=[
          pl.BlockSpec((bb, tm, D), lambda b, i: (b, i, 0)),
          pl.BlockSpec((bb, s_pad, D), lambda b, i: (b, 0, 0)),
          pl.BlockSpec((bb, tm, s_pad), lambda b, i: (b, i, 0)),
          pl.BlockSpec((D, 2 * D), lambda b, i: (0, 0)),
          pl.BlockSpec((1, 2 * D), lambda b, i: (0, 0)),
          pl.BlockSpec((1, 1), lambda b, i: (0, 0)),
      ],
      out_specs=[
          pl.BlockSpec((bb, tm, D), lambda b, i: (b, i, 0)),
          pl.BlockSpec((bb, tm, s_pad), lambda b, i: (b, i, 0)),
          pl.BlockSpec((bb, 1, tm), lambda b, i: (b, 0, i)),
      ],
      compiler_params=pltpu.CompilerParams(
          dimension_semantics=("parallel", "parallel"),
          vmem_limit_bytes=56 << 20),
  )(q, ctx, msk, jnp.asarray(w_attn, jnp.float32),
    jnp.asarray(w_switch, jnp.float32), b_sw)

  switch = switch.reshape(B, t_pad, 1)
  return out[:, :T, :], weight[:, :T, :S], switch[:, :T, :]


# speedup vs baseline: 1.4592x; 1.0129x over previous
import functools

import jax
import jax.numpy as jnp
from jax.experimental import pallas as pl
from jax.experimental.pallas import tpu as pltpu


def _pd_kernel(msk_ref, q_ref, ctx_ref, wattn_ref, wsw_ref, bsw_ref,
               out_ref, wgt_ref, sw_ref, *, D, bb):
  w_top = wattn_ref[:, :D].astype(jnp.bfloat16)
  w_bot = wattn_ref[:, D:].astype(jnp.bfloat16)

  for j in range(bb):
    q32 = q_ref[j]
    qb = q32.astype(jnp.bfloat16)
    cb = ctx_ref[j].astype(jnp.bfloat16)

    scores = jax.lax.dot_general(qb, cb, (((1,), (1,)), ((), ())),
                                 preferred_element_type=jnp.float32)
    scores = jnp.where(msk_ref[j] != 0.0, scores, jnp.float32(-1e30))

    m = jnp.max(scores, axis=-1, keepdims=True)
    p = jnp.exp(scores - m)
    l = jnp.sum(p, axis=-1, keepdims=True)
    inv_l = pl.reciprocal(l, approx=False)
    wgt_ref[j] = p * inv_l

    attn = jax.lax.dot_general(p.astype(jnp.bfloat16), cb,
                               (((1,), (0,)), ((), ())),
                               preferred_element_type=jnp.float32) * inv_l

    h = (jax.lax.dot_general(qb, w_top, (((1,), (1,)), ((), ())),
                             preferred_element_type=jnp.float32) +
         jax.lax.dot_general(attn.astype(jnp.bfloat16), w_bot,
                             (((1,), (1,)), ((), ())),
                             preferred_element_type=jnp.float32))
    out = jnp.tanh(h)
    out_ref[j] = out.astype(out_ref.dtype)

    z = (jax.lax.dot_general(wsw_ref[0:1, :D], q32, (((1,), (1,)), ((), ())),
                             preferred_element_type=jnp.float32) +
         jax.lax.dot_general(wsw_ref[0:1, D:], out, (((1,), (1,)), ((), ())),
                             preferred_element_type=jnp.float32) +
         bsw_ref[...])
    sw_ref[j] = jax.nn.sigmoid(z)


def _round_up(x, m):
  return ((x + m - 1) // m) * m


def kernel(inp, context, atten_mask, w_attn, w_switch, b_switch):
  B, T, D = inp.shape
  _, S, _ = context.shape

  b_sw = jnp.asarray(b_switch, jnp.float32).reshape(1, 1)

  bb = 2 if B % 2 == 0 else 1
  tm = min(512, _round_up(T, 8))
  t_pad, s_pad = _round_up(T, tm), _round_up(S, 128)
  nt = t_pad // tm

  q = jnp.asarray(inp, jnp.float32)
  ctx = jnp.asarray(context, jnp.float32)
  msk = jnp.asarray(atten_mask, jnp.float32)
  if t_pad != T:
    q = jnp.pad(q, ((0, 0), (0, t_pad - T), (0, 0)))
    msk = jnp.pad(msk, ((0, 0), (0, t_pad - T), (0, 0)))
  if s_pad != S:
    ctx = jnp.pad(ctx, ((0, 0), (0, s_pad - S), (0, 0)))
    msk = jnp.pad(msk, ((0, 0), (0, 0), (0, s_pad - S)))

  out_shapes = (
      jax.ShapeDtypeStruct((B, t_pad, D), jnp.float32),
      jax.ShapeDtypeStruct((B, t_pad, s_pad), jnp.float32),
      jax.ShapeDtypeStruct((B, 1, t_pad), jnp.float32),
  )

  kfn = functools.partial(_pd_kernel, D=D, bb=bb)

  out, weight, switch = pl.pallas_call(
      kfn,
      out_shape=out_shapes,
      grid=(B // bb, nt),
      in_specs=[
          pl.BlockSpec((bb, tm, s_pad), lambda b, i: (b, i, 0)),
          pl.BlockSpec((bb, tm, D), lambda b, i: (b, i, 0)),
          pl.BlockSpec((bb, s_pad, D), lambda b, i: (b, 0, 0)),
          pl.BlockSpec((D, 2 * D), lambda b, i: (0, 0)),
          pl.BlockSpec((1, 2 * D), lambda b, i: (0, 0)),
          pl.BlockSpec((1, 1), lambda b, i: (0, 0)),
      ],
      out_specs=[
          pl.BlockSpec((bb, tm, D), lambda b, i: (b, i, 0)),
          pl.BlockSpec((bb, tm, s_pad), lambda b, i: (b, i, 0)),
          pl.BlockSpec((bb, 1, tm), lambda b, i: (b, 0, i)),
      ],
      compiler_params=pltpu.CompilerParams(
          dimension_semantics=("parallel", "parallel"),
          vmem_limit_bytes=56 << 20),
  )(msk, q, ctx, jnp.asarray(w_attn, jnp.float32),
    jnp.asarray(w_switch, jnp.float32), b_sw)

  switch = switch.reshape(B, t_pad, 1)
  return out[:, :T, :], weight[:, :T, :S], switch[:, :T, :]
